# Initial kernel scaffold; baseline (speedup 1.0000x reference)
import jax
import jax.numpy as jnp
from jax.experimental import pallas as pl


def kernel(x, edge_index, W1, b1, W2, b2, Wc, bc):
    raise NotImplementedError("write your pallas kernel here")


# speedup vs baseline: 19.2976x; 19.2976x over previous
import functools

import jax
import jax.numpy as jnp
from jax import lax
from jax.experimental import pallas as pl
from jax.experimental.pallas import tpu as pltpu
from jax.experimental.pallas import tpu_sc as plsc

_NC = 2
_NS = 16
_NW = _NC * _NS
_C = 80


def _sc_degree(dst1, halves, n_pad):
    e = dst1.shape[0]
    epw = e // _NW
    niter = epw // _C
    rps = n_pad // _NS
    zc = rps // 2
    ring = 8
    mesh = plsc.VectorSubcoreMesh(core_axis_name="c", subcore_axis_name="s")

    @functools.partial(
        pl.kernel,
        out_type=jax.ShapeDtypeStruct((_NC, n_pad, 16), jnp.float32),
        mesh=mesh,
        scratch_types=[
            pltpu.VMEM((ring, _C), jnp.int32),
            pltpu.VMEM((_C, 16), jnp.float32),
            pltpu.VMEM_SHARED((n_pad, 16), jnp.float32),
            pltpu.SemaphoreType.DMA,
        ],
    )
    def k(dst_hbm, half_hbm, out_hbm, dv, ones_v, acc, dsem):
        ci = lax.axis_index("c")
        si = lax.axis_index("s")
        wid = si * _NC + ci
        base = wid * epw

        def fill_ones(i, _):
            ones_v[i, :] = jnp.full((16,), 1.0, jnp.float32)
            return 0

        lax.fori_loop(0, _C, fill_ones, 0)

        pltpu.sync_copy(half_hbm, acc.at[pl.ds(si * rps, rps)])
        plsc.subcore_barrier()

        def body(j, _):
            slot = lax.rem(j, ring)

            @pl.when(j >= ring)
            def _():
                pltpu.make_async_copy(
                    ones_v, acc.at[dv.at[slot]], dsem).wait()

            pltpu.sync_copy(dst_hbm.at[pl.ds(base + j * _C, _C)],
                            dv.at[slot])
            pltpu.async_copy(ones_v, acc.at[dv.at[slot]], dsem, add=True)
            return 0

        lax.fori_loop(0, niter, body, 0)

        def drain(j, _):
            slot = lax.rem(j, ring)
            pltpu.make_async_copy(ones_v, acc.at[dv.at[slot]], dsem).wait()
            return 0

        lax.fori_loop(niter - ring, niter, drain, 0)
        plsc.subcore_barrier()
        pltpu.sync_copy(acc.at[pl.ds(si * rps, rps)],
                        out_hbm.at[ci, pl.ds(si * rps, rps)])

    return k(dst1, halves)


def _sc_scatter(g, src1, dst1, zrows, n_pad, h):
    e = src1.shape[0]
    epw = e // _NW
    niter = epw // _C
    rps = n_pad // _NS
    mesh = plsc.VectorSubcoreMesh(core_axis_name="c", subcore_axis_name="s")

    @functools.partial(
        pl.kernel,
        out_type=jax.ShapeDtypeStruct((_NC, n_pad, h), jnp.float32),
        mesh=mesh,
        scratch_types=[
            pltpu.VMEM((2, _C), jnp.int32),
            pltpu.VMEM((2, _C), jnp.int32),
            pltpu.VMEM((_C, h), jnp.float32),
            pltpu.VMEM((_C, h), jnp.float32),
            pltpu.VMEM_SHARED((n_pad, h), jnp.float32),
            pltpu.SemaphoreType.DMA,
            pltpu.SemaphoreType.DMA,
        ],
    )
    def k(g_hbm, src_hbm, dst_hbm, z_hbm, out_hbm, sv, dv, rb0, rb1,
          acc, gsem0, gsem1):
        ci = lax.axis_index("c")
        si = lax.axis_index("s")
        wid = si * _NC + ci
        base = wid * epw

        @pl.when(ci == 0)
        def _():
            pltpu.sync_copy(g_hbm.at[pl.ds(si * rps, rps)],
                            acc.at[pl.ds(si * rps, rps)])

        @pl.when(ci != 0)
        def _():
            pltpu.sync_copy(z_hbm, acc.at[pl.ds(si * rps, rps)])

        plsc.subcore_barrier()

        def load_idx(j, parity):
            pltpu.sync_copy(src_hbm.at[pl.ds(base + j * _C, _C)],
                            sv.at[parity])
            pltpu.sync_copy(dst_hbm.at[pl.ds(base + j * _C, _C)],
                            dv.at[parity])

        load_idx(0, 0)
        pltpu.async_copy(g_hbm.at[sv.at[0]], rb0, gsem0)

        def body(i, _):
            load_idx(2 * i + 1, 1)
            pltpu.async_copy(g_hbm.at[sv.at[1]], rb1, gsem1)
            pltpu.make_async_copy(g_hbm.at[sv.at[0]], rb0, gsem0).wait()
            pltpu.sync_copy(rb0, acc.at[dv.at[0]], add=True)
            load_idx(2 * i + 2, 0)
            pltpu.async_copy(g_hbm.at[sv.at[0]], rb0, gsem0)
            pltpu.make_async_copy(g_hbm.at[sv.at[1]], rb1, gsem1).wait()
            pltpu.sync_copy(rb1, acc.at[dv.at[1]], add=True)
            return 0

        lax.fori_loop(0, (niter - 1) // 2, body, 0)
        pltpu.make_async_copy(g_hbm.at[sv.at[0]], rb0, gsem0).wait()
        pltpu.sync_copy(rb0, acc.at[dv.at[0]], add=True)

        plsc.subcore_barrier()
        pltpu.sync_copy(acc.at[pl.ds(si * rps, rps)],
                        out_hbm.at[ci, pl.ds(si * rps, rps)])

    return k(g, src1, dst1, zrows)


def _dinv_block(deg_ref):
    deg = deg_ref[0, :, 0:1] + deg_ref[1, :, 0:1]
    return lax.rsqrt(deg)


def _tc_first(deg_p, x, w1, bn):
    n_pad = deg_p.shape[1]
    d = x.shape[1]
    h = w1.shape[1]

    def body(deg_ref, x_ref, w_ref, g_ref):
        dinv = _dinv_block(deg_ref)
        hm = jnp.dot(x_ref[...], w_ref[...], preferred_element_type=jnp.float32)
        g_ref[...] = hm * dinv

    return pl.pallas_call(
        body,
        grid=(n_pad // bn,),
        in_specs=[
            pl.BlockSpec((_NC, bn, 16), lambda i: (0, i, 0)),
            pl.BlockSpec((bn, d), lambda i: (i, 0)),
            pl.BlockSpec((d, h), lambda i: (0, 0)),
        ],
        out_specs=pl.BlockSpec((bn, h), lambda i: (i, 0)),
        out_shape=jax.ShapeDtypeStruct((n_pad, h), jnp.float32),
    )(deg_p, x, w1)


def _tc_mid(deg_p, p, b, w2, bn):
    _, n_pad, h = p.shape
    h2 = w2.shape[1]

    def body(deg_ref, p_ref, b_ref, w_ref, g_ref):
        dinv = _dinv_block(deg_ref)
        f = jnp.maximum((p_ref[0] + p_ref[1]) * dinv + b_ref[...], 0.0)
        g_ref[...] = jnp.dot(f, w_ref[...],
                             preferred_element_type=jnp.float32) * dinv

    return pl.pallas_call(
        body,
        grid=(n_pad // bn,),
        in_specs=[
            pl.BlockSpec((_NC, bn, 16), lambda i: (0, i, 0)),
            pl.BlockSpec((_NC, bn, h), lambda i: (0, i, 0)),
            pl.BlockSpec((1, h), lambda i: (0, 0)),
            pl.BlockSpec((h, h2), lambda i: (0, 0)),
        ],
        out_specs=pl.BlockSpec((bn, h2), lambda i: (i, 0)),
        out_shape=jax.ShapeDtypeStruct((n_pad, h2), jnp.float32),
    )(deg_p, p, b, w2)


def _tc_last(deg_p, p, b, wc, bc, n, bn):
    _, n_pad, h = p.shape
    o = wc.shape[1]

    def body(deg_ref, p_ref, b_ref, w_ref, bc_ref, o_ref):
        dinv = _dinv_block(deg_ref)
        f = jnp.maximum((p_ref[0] + p_ref[1]) * dinv + b_ref[...], 0.0)
        o_ref[...] = jnp.dot(f, w_ref[...],
                             preferred_element_type=jnp.float32) + bc_ref[...]

    return pl.pallas_call(
        body,
        grid=(n // bn,),
        in_specs=[
            pl.BlockSpec((_NC, bn, 16), lambda i: (0, i, 0)),
            pl.BlockSpec((_NC, bn, h), lambda i: (0, i, 0)),
            pl.BlockSpec((1, h), lambda i: (0, 0)),
            pl.BlockSpec((h, o), lambda i: (0, 0)),
            pl.BlockSpec((1, o), lambda i: (0, 0)),
        ],
        out_specs=pl.BlockSpec((bn, o), lambda i: (i, 0)),
        out_shape=jax.ShapeDtypeStruct((n, o), jnp.float32),
    )(deg_p, p, b, wc, bc)


def kernel(x, edge_index, W1, b1, W2, b2, Wc, bc):
    n, d = x.shape
    e = edge_index.shape[1]
    h = W1.shape[1]
    epw = e // _NW
    niter = epw // _C
    n_pad = ((n + _NS * 8 - 1) // (_NS * 8)) * (_NS * 8)
    assert epw * _NW == e and niter * _C == epw and niter % 2 == 1

    src1 = edge_index[0]
    dst1 = edge_index[1]
    zrows = jnp.zeros((n_pad // _NS, h), jnp.float32)
    halves = jnp.full((n_pad // _NS, 16), 0.5, jnp.float32)

    deg_p = _sc_degree(dst1, halves, n_pad)
    g1 = _tc_first(deg_p, x, W1, bn=n_pad // 16)
    p1 = _sc_scatter(g1, src1, dst1, zrows, n_pad, h)
    g2 = _tc_mid(deg_p, p1, b1.reshape(1, -1), W2, bn=n_pad // 16)
    p2 = _sc_scatter(g2, src1, dst1, zrows, n_pad, h)
    return _tc_last(deg_p, p2, b2.reshape(1, -1), Wc, bc.reshape(1, -1),
                    n, bn=1000)


# speedup vs baseline: 29.0545x; 1.5056x over previous
import functools

import jax
import jax.numpy as jnp
from jax import lax
from jax.experimental import pallas as pl
from jax.experimental.pallas import tpu as pltpu
from jax.experimental.pallas import tpu_sc as plsc

_NC = 2
_NS = 16
_NW = _NC * _NS
_C = 80


def _sc_degree(dst1, halves, n_pad):
    e = dst1.shape[0]
    epw = e // _NW
    niter = epw // _C
    rps = n_pad // _NS
    zc = rps // 2
    ring = 8
    mesh = plsc.VectorSubcoreMesh(core_axis_name="c", subcore_axis_name="s")

    @functools.partial(
        pl.kernel,
        out_type=jax.ShapeDtypeStruct((_NC, n_pad, 16), jnp.float32),
        mesh=mesh,
        scratch_types=[
            pltpu.VMEM((ring, _C), jnp.int32),
            pltpu.VMEM((epw,), jnp.int32),
            pltpu.VMEM((_C, 16), jnp.float32),
            pltpu.VMEM_SHARED((n_pad, 16), jnp.float32),
            pltpu.SemaphoreType.DMA,
        ],
    )
    def k(dst_hbm, half_hbm, out_hbm, dv, dv1, ones_v, acc, dsem):
        ci = lax.axis_index("c")
        si = lax.axis_index("s")
        wid = si * _NC + ci
        base = wid * epw

        def fill_ones(i, _):
            ones_v[i, :] = jnp.full((16,), 1.0, jnp.float32)
            return 0

        lax.fori_loop(0, _C, fill_ones, 0)

        pltpu.sync_copy(half_hbm, acc.at[pl.ds(si * rps, rps)])
        pltpu.sync_copy(dst_hbm.at[pl.ds(base, epw)], dv1)
        plsc.subcore_barrier()

        def body(j, _):
            slot = lax.rem(j, ring)

            @pl.when(j >= ring)
            def _():
                pltpu.make_async_copy(
                    ones_v, acc.at[dv.at[slot]], dsem).wait()

            for q in range(_C // 16):
                dv[slot, pl.ds(q * 16, 16)] = dv1[pl.ds(j * _C + q * 16, 16)]
            pltpu.async_copy(ones_v, acc.at[dv.at[slot]], dsem, add=True)
            return 0

        lax.fori_loop(0, niter, body, 0)

        def drain(j, _):
            slot = lax.rem(j, ring)
            pltpu.make_async_copy(ones_v, acc.at[dv.at[slot]], dsem).wait()
            return 0

        lax.fori_loop(niter - ring, niter, drain, 0)
        plsc.subcore_barrier()
        pltpu.sync_copy(acc.at[pl.ds(si * rps, rps)],
                        out_hbm.at[ci, pl.ds(si * rps, rps)])

    return k(dst1, halves)


def _sc_scatter(g, src1, dst1, zrows, n_pad, h):
    e = src1.shape[0]
    epw = e // _NW
    niter = epw // _C
    rps = n_pad // _NS
    mesh = plsc.VectorSubcoreMesh(core_axis_name="c", subcore_axis_name="s")

    @functools.partial(
        pl.kernel,
        out_type=jax.ShapeDtypeStruct((_NC, n_pad, h), jnp.float32),
        mesh=mesh,
        scratch_types=[
            pltpu.VMEM((epw,), jnp.int32),
            pltpu.VMEM((epw,), jnp.int32),
            pltpu.VMEM((2, _C), jnp.int32),
            pltpu.VMEM((_C, h), jnp.float32),
            pltpu.VMEM((_C, h), jnp.float32),
            pltpu.VMEM_SHARED((n_pad, h), jnp.float32),
            pltpu.SemaphoreType.DMA,
            pltpu.SemaphoreType.DMA,
        ],
    )
    def k(g_hbm, src_hbm, dst_hbm, z_hbm, out_hbm, sv1, dv1, ds2, rb0, rb1,
          acc, gsem0, gsem1):
        ci = lax.axis_index("c")
        si = lax.axis_index("s")
        wid = si * _NC + ci
        base = wid * epw

        @pl.when(ci == 0)
        def _():
            pltpu.sync_copy(g_hbm.at[pl.ds(si * rps, rps)],
                            acc.at[pl.ds(si * rps, rps)])

        @pl.when(ci != 0)
        def _():
            pltpu.sync_copy(z_hbm, acc.at[pl.ds(si * rps, rps)])

        pltpu.sync_copy(src_hbm.at[pl.ds(base, epw)], sv1)
        pltpu.sync_copy(dst_hbm.at[pl.ds(base, epw)], dv1)
        plsc.subcore_barrier()

        def gidx(j):
            return sv1.at[pl.ds(j * _C, _C)]

        def stage_didx(j, parity):
            for q in range(_C // 16):
                ds2[parity, pl.ds(q * 16, 16)] = dv1[pl.ds(j * _C + q * 16,
                                                           16)]

        pltpu.async_copy(g_hbm.at[gidx(0)], rb0, gsem0)

        def body(i, _):
            j0 = 2 * i
            pltpu.async_copy(g_hbm.at[gidx(j0 + 1)], rb1, gsem1)
            stage_didx(j0, 0)
            pltpu.make_async_copy(g_hbm.at[gidx(j0)], rb0, gsem0).wait()
            pltpu.sync_copy(rb0, acc.at[ds2.at[0]], add=True)
            pltpu.async_copy(g_hbm.at[gidx(j0 + 2)], rb0, gsem0)
            stage_didx(j0 + 1, 1)
            pltpu.make_async_copy(g_hbm.at[gidx(j0 + 1)], rb1, gsem1).wait()
            pltpu.sync_copy(rb1, acc.at[ds2.at[1]], add=True)
            return 0

        lax.fori_loop(0, (niter - 1) // 2, body, 0)
        stage_didx(niter - 1, 0)
        pltpu.make_async_copy(g_hbm.at[gidx(niter - 1)], rb0, gsem0).wait()
        pltpu.sync_copy(rb0, acc.at[ds2.at[0]], add=True)

        plsc.subcore_barrier()
        pltpu.sync_copy(acc.at[pl.ds(si * rps, rps)],
                        out_hbm.at[ci, pl.ds(si * rps, rps)])

    return k(g, src1, dst1, zrows)


def _dinv_block(deg_ref):
    deg = deg_ref[0, :, 0:1] + deg_ref[1, :, 0:1]
    return lax.rsqrt(deg)


def _tc_first(deg_p, x, w1, bn):
    n_pad = deg_p.shape[1]
    d = x.shape[1]
    h = w1.shape[1]

    def body(deg_ref, x_ref, w_ref, g_ref):
        dinv = _dinv_block(deg_ref)
        hm = jnp.dot(x_ref[...], w_ref[...], preferred_element_type=jnp.float32)
        g_ref[...] = hm * dinv

    return pl.pallas_call(
        body,
        grid=(n_pad // bn,),
        in_specs=[
            pl.BlockSpec((_NC, bn, 16), lambda i: (0, i, 0)),
            pl.BlockSpec((bn, d), lambda i: (i, 0)),
            pl.BlockSpec((d, h), lambda i: (0, 0)),
        ],
        out_specs=pl.BlockSpec((bn, h), lambda i: (i, 0)),
        out_shape=jax.ShapeDtypeStruct((n_pad, h), jnp.float32),
    )(deg_p, x, w1)


def _tc_mid(deg_p, p, b, w2, bn):
    _, n_pad, h = p.shape
    h2 = w2.shape[1]

    def body(deg_ref, p_ref, b_ref, w_ref, g_ref):
        dinv = _dinv_block(deg_ref)
        f = jnp.maximum((p_ref[0] + p_ref[1]) * dinv + b_ref[...], 0.0)
        g_ref[...] = jnp.dot(f, w_ref[...],
                             preferred_element_type=jnp.float32) * dinv

    return pl.pallas_call(
        body,
        grid=(n_pad // bn,),
        in_specs=[
            pl.BlockSpec((_NC, bn, 16), lambda i: (0, i, 0)),
            pl.BlockSpec((_NC, bn, h), lambda i: (0, i, 0)),
            pl.BlockSpec((1, h), lambda i: (0, 0)),
            pl.BlockSpec((h, h2), lambda i: (0, 0)),
        ],
        out_specs=pl.BlockSpec((bn, h2), lambda i: (i, 0)),
        out_shape=jax.ShapeDtypeStruct((n_pad, h2), jnp.float32),
    )(deg_p, p, b, w2)


def _tc_last(deg_p, p, b, wc, bc, n, bn):
    _, n_pad, h = p.shape
    o = wc.shape[1]

    def body(deg_ref, p_ref, b_ref, w_ref, bc_ref, o_ref):
        dinv = _dinv_block(deg_ref)
        f = jnp.maximum((p_ref[0] + p_ref[1]) * dinv + b_ref[...], 0.0)
        o_ref[...] = jnp.dot(f, w_ref[...],
                             preferred_element_type=jnp.float32) + bc_ref[...]

    return pl.pallas_call(
        body,
        grid=(n // bn,),
        in_specs=[
            pl.BlockSpec((_NC, bn, 16), lambda i: (0, i, 0)),
            pl.BlockSpec((_NC, bn, h), lambda i: (0, i, 0)),
            pl.BlockSpec((1, h), lambda i: (0, 0)),
            pl.BlockSpec((h, o), lambda i: (0, 0)),
            pl.BlockSpec((1, o), lambda i: (0, 0)),
        ],
        out_specs=pl.BlockSpec((bn, o), lambda i: (i, 0)),
        out_shape=jax.ShapeDtypeStruct((n, o), jnp.float32),
    )(deg_p, p, b, wc, bc)


def kernel(x, edge_index, W1, b1, W2, b2, Wc, bc):
    n, d = x.shape
    e = edge_index.shape[1]
    h = W1.shape[1]
    epw = e // _NW
    niter = epw // _C
    n_pad = ((n + _NS * 8 - 1) // (_NS * 8)) * (_NS * 8)
    assert epw * _NW == e and niter * _C == epw and niter % 2 == 1

    src1 = edge_index[0]
    dst1 = edge_index[1]
    zrows = jnp.zeros((n_pad // _NS, h), jnp.float32)
    halves = jnp.full((n_pad // _NS, 16), 0.5, jnp.float32)

    deg_p = _sc_degree(dst1, halves, n_pad)
    g1 = _tc_first(deg_p, x, W1, bn=n_pad // 16)
    p1 = _sc_scatter(g1, src1, dst1, zrows, n_pad, h)
    g2 = _tc_mid(deg_p, p1, b1.reshape(1, -1), W2, bn=n_pad // 16)
    p2 = _sc_scatter(g2, src1, dst1, zrows, n_pad, h)
    return _tc_last(deg_p, p2, b2.reshape(1, -1), Wc, bc.reshape(1, -1),
                    n, bn=1000)
